# Initial kernel scaffold; baseline (speedup 1.0000x reference)
import jax
import jax.numpy as jnp
from jax.experimental import pallas as pl


def kernel(input, target):
    raise NotImplementedError("write your pallas kernel here")


# speedup vs baseline: 27.0107x; 27.0107x over previous
import functools

import jax
import jax.numpy as jnp
from jax import lax
from jax.experimental import pallas as pl
from jax.experimental.pallas import tpu as pltpu
from jax.experimental.pallas import tpu_sc as plsc

_K = 150
_CPAD = 160
_NBINS = _K * _CPAD
_NC, _NS, _L = 2, 16, 16
_NW = _NC * _NS
_NPIX = 16 * 512 * 512
_PER_W = _NPIX // _NW
_CHUNK = 8192
_NCHUNKS = _PER_W // _CHUNK
_VECS = _CHUNK // _L

_mesh = plsc.VectorSubcoreMesh(
    core_axis_name="c", subcore_axis_name="s", num_cores=_NC, num_subcores=_NS
)


@functools.partial(
    pl.kernel,
    out_type=jax.ShapeDtypeStruct((_NW, _K, _CPAD), jnp.int32),
    mesh=_mesh,
    scratch_types=[
        pltpu.VMEM((_K, _CPAD), jnp.int32),
        pltpu.VMEM((_CHUNK,), jnp.int32),
        pltpu.VMEM((_CHUNK,), jnp.int32),
        pltpu.SemaphoreType.DMA,
        pltpu.SemaphoreType.DMA,
    ],
    compiler_params=pltpu.CompilerParams(needs_layout_passes=False),
)
def _hist_sc(p_hbm, t_hbm, out_hbm, hist, p_buf, t_buf, sem_p, sem_t):
    wid = lax.axis_index("s") * _NC + lax.axis_index("c")

    zeros = jnp.zeros((_L,), jnp.int32)

    def zbody(i, carry):
        hist[i // (_CPAD // _L), pl.ds((i % (_CPAD // _L)) * _L, _L)] = zeros
        return carry

    lax.fori_loop(0, _NBINS // _L, zbody, 0)

    ones = jnp.ones((_L,), jnp.int32)
    base_w = wid * _PER_W

    def chunk_body(c, carry):
        base = base_w + c * _CHUNK
        cp_p = pltpu.async_copy(p_hbm.at[pl.ds(base, _CHUNK)], p_buf, sem_p)
        cp_t = pltpu.async_copy(t_hbm.at[pl.ds(base, _CHUNK)], t_buf, sem_t)
        cp_p.wait()
        cp_t.wait()

        def vbody(i, inner):
            pv = p_buf[pl.ds(i * _L, _L)]
            tv = t_buf[pl.ds(i * _L, _L)]
            plsc.addupdate_scatter(hist, [tv, pv], ones)
            return inner

        lax.fori_loop(0, _VECS, vbody, 0)
        return carry

    lax.fori_loop(0, _NCHUNKS, chunk_body, 0)
    pltpu.sync_copy(hist, out_hbm.at[wid])


def _iou_tc(hist_ref, out_ref):
    h = hist_ref[...]
    conf = h.sum(0).astype(jnp.float32)
    rows = lax.broadcasted_iota(jnp.int32, (_K, _CPAD), 0)
    cols = lax.broadcasted_iota(jnp.int32, (_K, _CPAD), 1)
    eye = rows == cols
    diag = jnp.where(eye, conf, 0.0).sum(1, keepdims=True)
    rowsum = conf.sum(1, keepdims=True)
    ct = conf.sum(0, keepdims=True)
    colsum = jnp.where(eye, jnp.broadcast_to(ct, conf.shape), 0.0).sum(
        1, keepdims=True
    )
    iou = diag / (rowsum + colsum - diag)
    out_ref[...] = (iou.sum() / _K).reshape(1, 1)


def kernel(input, target):
    p = input.reshape(-1)
    t = target.reshape(-1)
    partials = _hist_sc(p, t)
    res = pl.pallas_call(
        _iou_tc,
        out_shape=jax.ShapeDtypeStruct((1, 1), jnp.float32),
    )(partials)
    return res[0, 0]


# speedup vs baseline: 39.8283x; 1.4745x over previous
import functools

import jax
import jax.numpy as jnp
from jax import lax
from jax.experimental import pallas as pl
from jax.experimental.pallas import tpu as pltpu
from jax.experimental.pallas import tpu_sc as plsc

_K = 150
_CPAD = 160
_NC, _NS, _L = 2, 16, 16
_NW = _NC * _NS
_B, _H, _W = 16, 512, 512
_HALF = _H // 2
_ROWS = 32
_NCH = _HALF // _ROWS
_VPR = _W // _L

_mesh = plsc.VectorSubcoreMesh(
    core_axis_name="c", subcore_axis_name="s", num_cores=_NC, num_subcores=_NS
)


@functools.partial(
    pl.kernel,
    out_type=jax.ShapeDtypeStruct((_NW, _K, _CPAD), jnp.int32),
    mesh=_mesh,
    scratch_types=[
        pltpu.VMEM((_K, _CPAD), jnp.int32),
        pltpu.VMEM((_ROWS, _W), jnp.int32),
        pltpu.VMEM((_ROWS, _W), jnp.int32),
        pltpu.VMEM((_ROWS, _W), jnp.int32),
        pltpu.VMEM((_ROWS, _W), jnp.int32),
        pltpu.SemaphoreType.DMA,
        pltpu.SemaphoreType.DMA,
    ],
    compiler_params=pltpu.CompilerParams(needs_layout_passes=False),
)
def _hist_sc(p_hbm, t_hbm, out_hbm, hist, p0, p1, t0, t1, s0, s1):
    wid = lax.axis_index("s") * _NC + lax.axis_index("c")
    img = wid // 2
    row_base = (wid % 2) * _HALF

    zeros = jnp.zeros((_L,), jnp.int32)

    def zbody(r, carry):
        for j in range(_CPAD // _L):
            hist[r, pl.ds(j * _L, _L)] = zeros
        return carry

    lax.fori_loop(0, _K, zbody, 0)

    ones = jnp.ones((_L,), jnp.int32)
    pbufs, tbufs, sems = [p0, p1], [t0, t1], [s0, s1]

    def start(c):
        r0 = row_base + c * _ROWS
        cp = pltpu.async_copy(p_hbm.at[img, pl.ds(r0, _ROWS)], pbufs[c % 2], sems[c % 2])
        ct = pltpu.async_copy(t_hbm.at[img, pl.ds(r0, _ROWS)], tbufs[c % 2], sems[c % 2])
        return cp, ct

    pending = start(0)
    for c in range(_NCH):
        cp, ct = pending
        cp.wait()
        ct.wait()
        if c + 1 < _NCH:
            pending = start(c + 1)
        pb, tb = pbufs[c % 2], tbufs[c % 2]

        def row_body(r, carry, pb=pb, tb=tb):
            for j in range(_VPR):
                pv = pb[r, pl.ds(j * _L, _L)]
                tv = tb[r, pl.ds(j * _L, _L)]
                plsc.addupdate_scatter(hist, [tv, pv], ones)
            return carry

        lax.fori_loop(0, _ROWS, row_body, 0)

    pltpu.sync_copy(hist, out_hbm.at[wid])


def _iou_tc(hist_ref, out_ref):
    h = hist_ref[...]
    conf = h.sum(0).astype(jnp.float32)
    rows = lax.broadcasted_iota(jnp.int32, (_K, _CPAD), 0)
    cols = lax.broadcasted_iota(jnp.int32, (_K, _CPAD), 1)
    eye = rows == cols
    diag = jnp.where(eye, conf, 0.0).sum(1, keepdims=True)
    rowsum = conf.sum(1, keepdims=True)
    ct = conf.sum(0, keepdims=True)
    colsum = jnp.where(eye, jnp.broadcast_to(ct, conf.shape), 0.0).sum(
        1, keepdims=True
    )
    iou = diag / (rowsum + colsum - diag)
    out_ref[...] = (iou.sum() / _K).reshape(1, 1)


def kernel(input, target):
    partials = _hist_sc(input, target)
    res = pl.pallas_call(
        _iou_tc,
        out_shape=jax.ShapeDtypeStruct((1, 1), jnp.float32),
    )(partials)
    return res[0, 0]


# speedup vs baseline: 70.3567x; 1.7665x over previous
import functools

import jax
import jax.numpy as jnp
from jax import lax
from jax.experimental import pallas as pl
from jax.experimental.pallas import tpu as pltpu
from jax.experimental.pallas import tpu_sc as plsc

_K = 150
_CPAD = 160
_NC, _NS, _L = 2, 16, 16
_NW = _NC * _NS
_B, _H, _W = 16, 512, 512
_HALF = _H // 2
_ROWS = 32
_NCH = _HALF // _ROWS
_VPR = _W // _L

_mesh = plsc.VectorSubcoreMesh(
    core_axis_name="c", subcore_axis_name="s", num_cores=_NC, num_subcores=_NS
)


@functools.partial(
    pl.kernel,
    out_type=jax.ShapeDtypeStruct((_NW, _K, _CPAD), jnp.int32),
    mesh=_mesh,
    scratch_types=[
        pltpu.VMEM((_K, _CPAD), jnp.int32),
        pltpu.VMEM((_ROWS, _W), jnp.int32),
        pltpu.VMEM((_ROWS, _W), jnp.int32),
        pltpu.VMEM((_ROWS, _W), jnp.int32),
        pltpu.VMEM((_ROWS, _W), jnp.int32),
        pltpu.SemaphoreType.DMA,
        pltpu.SemaphoreType.DMA,
    ],
    compiler_params=pltpu.CompilerParams(needs_layout_passes=False),
)
def _hist_sc(p_hbm, t_hbm, out_hbm, hist, p0, p1, t0, t1, s0, s1):
    wid = lax.axis_index("s") * _NC + lax.axis_index("c")
    img = wid // 2
    row_base = (wid % 2) * _HALF

    zeros = jnp.zeros((_L,), jnp.int32)

    def zbody(r, carry):
        for j in range(_CPAD // _L):
            hist[r, pl.ds(j * _L, _L)] = zeros
        return carry

    lax.fori_loop(0, _K, zbody, 0)

    ones = jnp.ones((_L,), jnp.int32)
    pbufs, tbufs, sems = [p0, p1], [t0, t1], [s0, s1]

    def start(c):
        r0 = row_base + c * _ROWS
        cp = pltpu.async_copy(p_hbm.at[img, pl.ds(r0, _ROWS)], pbufs[c % 2], sems[c % 2])
        ct = pltpu.async_copy(t_hbm.at[img, pl.ds(r0, _ROWS)], tbufs[c % 2], sems[c % 2])
        return cp, ct

    pending = start(0)
    for c in range(_NCH):
        cp, ct = pending
        cp.wait()
        ct.wait()
        if c + 1 < _NCH:
            pending = start(c + 1)
        pb, tb = pbufs[c % 2], tbufs[c % 2]

        def row_body(r, pb=pb, tb=tb):
            for j in range(_VPR):
                pv = pb[r, pl.ds(j * _L, _L)]
                tv = tb[r, pl.ds(j * _L, _L)]
                plsc.addupdate_scatter(hist, [tv, pv], ones)

        plsc.parallel_loop(0, _ROWS)(row_body)

    pltpu.sync_copy(hist, out_hbm.at[wid])


def _iou_tc(hist_ref, out_ref):
    h = hist_ref[...]
    conf = h.sum(0).astype(jnp.float32)
    rows = lax.broadcasted_iota(jnp.int32, (_K, _CPAD), 0)
    cols = lax.broadcasted_iota(jnp.int32, (_K, _CPAD), 1)
    eye = rows == cols
    diag = jnp.where(eye, conf, 0.0).sum(1, keepdims=True)
    rowsum = conf.sum(1, keepdims=True)
    ct = conf.sum(0, keepdims=True)
    colsum = jnp.where(eye, jnp.broadcast_to(ct, conf.shape), 0.0).sum(
        1, keepdims=True
    )
    iou = diag / (rowsum + colsum - diag)
    out_ref[...] = (iou.sum() / _K).reshape(1, 1)


def kernel(input, target):
    partials = _hist_sc(input, target)
    res = pl.pallas_call(
        _iou_tc,
        out_shape=jax.ShapeDtypeStruct((1, 1), jnp.float32),
    )(partials)
    return res[0, 0]


# speedup vs baseline: 71.6101x; 1.0178x over previous
import functools

import jax
import jax.numpy as jnp
from jax import lax
from jax.experimental import pallas as pl
from jax.experimental.pallas import tpu as pltpu
from jax.experimental.pallas import tpu_sc as plsc

_K = 150
_CPAD = 160
_NC, _NS, _L = 2, 16, 16
_NW = _NC * _NS
_B, _H, _W = 16, 512, 512
_HALF = _H // 2
_ROWS = 32
_NCH = _HALF // _ROWS
_VPR = _W // _L

_mesh = plsc.VectorSubcoreMesh(
    core_axis_name="c", subcore_axis_name="s", num_cores=_NC, num_subcores=_NS
)


@functools.partial(
    pl.kernel,
    out_type=jax.ShapeDtypeStruct((_NW, _K * _CPAD), jnp.int32),
    mesh=_mesh,
    scratch_types=[
        pltpu.VMEM((_K * _CPAD,), jnp.int32),
        pltpu.VMEM((_ROWS, _W), jnp.int32),
        pltpu.VMEM((_ROWS, _W), jnp.int32),
        pltpu.VMEM((_ROWS, _W), jnp.int32),
        pltpu.VMEM((_ROWS, _W), jnp.int32),
        pltpu.SemaphoreType.DMA,
        pltpu.SemaphoreType.DMA,
    ],
    compiler_params=pltpu.CompilerParams(needs_layout_passes=False),
)
def _hist_sc(p_hbm, t_hbm, out_hbm, hist, p0, p1, t0, t1, s0, s1):
    wid = lax.axis_index("s") * _NC + lax.axis_index("c")
    img = wid // 2
    row_base = (wid % 2) * _HALF

    zeros = jnp.zeros((_L,), jnp.int32)

    def zbody(r, carry):
        for j in range(_CPAD // _L):
            hist[pl.ds(r * _CPAD + j * _L, _L)] = zeros
        return carry

    lax.fori_loop(0, _K, zbody, 0)

    ones = jnp.ones((_L,), jnp.int32)
    pbufs, tbufs, sems = [p0, p1], [t0, t1], [s0, s1]

    def start(c):
        r0 = row_base + c * _ROWS
        cp = pltpu.async_copy(p_hbm.at[img, pl.ds(r0, _ROWS)], pbufs[c % 2], sems[c % 2])
        ct = pltpu.async_copy(t_hbm.at[img, pl.ds(r0, _ROWS)], tbufs[c % 2], sems[c % 2])
        return cp, ct

    pending = start(0)
    for c in range(_NCH):
        cp, ct = pending
        cp.wait()
        ct.wait()
        if c + 1 < _NCH:
            pending = start(c + 1)
        pb, tb = pbufs[c % 2], tbufs[c % 2]

        def row_body(r, pb=pb, tb=tb):
            for j in range(_VPR):
                pv = pb[r, pl.ds(j * _L, _L)]
                tv = tb[r, pl.ds(j * _L, _L)]
                plsc.addupdate_scatter(hist, [tv * _CPAD + pv], ones)

        plsc.parallel_loop(0, _ROWS)(row_body)

    pltpu.sync_copy(hist, out_hbm.at[wid])


def _iou_tc(hist_ref, out_ref):
    h = hist_ref[...]
    conf = h.sum(0).astype(jnp.float32)
    rows = lax.broadcasted_iota(jnp.int32, (_K, _CPAD), 0)
    cols = lax.broadcasted_iota(jnp.int32, (_K, _CPAD), 1)
    eye = rows == cols
    diag = jnp.where(eye, conf, 0.0).sum(1, keepdims=True)
    rowsum = conf.sum(1, keepdims=True)
    ct = conf.sum(0, keepdims=True)
    colsum = jnp.where(eye, jnp.broadcast_to(ct, conf.shape), 0.0).sum(
        1, keepdims=True
    )
    iou = diag / (rowsum + colsum - diag)
    out_ref[...] = (iou.sum() / _K).reshape(1, 1)


def kernel(input, target):
    partials = _hist_sc(input, target).reshape(_NW, _K, _CPAD)
    res = pl.pallas_call(
        _iou_tc,
        out_shape=jax.ShapeDtypeStruct((1, 1), jnp.float32),
    )(partials)
    return res[0, 0]


# speedup vs baseline: 74.8312x; 1.0450x over previous
import functools

import jax
import jax.numpy as jnp
from jax import lax
from jax.experimental import pallas as pl
from jax.experimental.pallas import tpu as pltpu
from jax.experimental.pallas import tpu_sc as plsc

_K = 150
_CPAD = 160
_NC, _NS, _L = 2, 16, 16
_NW = _NC * _NS
_B, _H, _W = 16, 512, 512
_HALF = _H // 2
_ROWS = 32
_NCH = _HALF // _ROWS
_VPR = _W // _L

_mesh = plsc.VectorSubcoreMesh(
    core_axis_name="c", subcore_axis_name="s", num_cores=_NC, num_subcores=_NS
)


@functools.partial(
    pl.kernel,
    out_type=jax.ShapeDtypeStruct((_NW, _K, _CPAD), jnp.int32),
    mesh=_mesh,
    scratch_types=[
        pltpu.VMEM((_K * _CPAD,), jnp.int32),
        pltpu.VMEM((_K, _CPAD), jnp.int32),
        pltpu.VMEM((_ROWS, _W), jnp.int32),
        pltpu.VMEM((_ROWS, _W), jnp.int32),
        pltpu.VMEM((_ROWS, _W), jnp.int32),
        pltpu.VMEM((_ROWS, _W), jnp.int32),
        pltpu.SemaphoreType.DMA,
        pltpu.SemaphoreType.DMA,
    ],
    compiler_params=pltpu.CompilerParams(needs_layout_passes=False),
)
def _hist_sc(p_hbm, t_hbm, out_hbm, hist, hist2d, p0, p1, t0, t1, s0, s1):
    wid = lax.axis_index("s") * _NC + lax.axis_index("c")
    img = wid // 2
    row_base = (wid % 2) * _HALF

    ones = jnp.ones((_L,), jnp.int32)
    zeros = jnp.zeros((_L,), jnp.int32)
    pbufs, tbufs, sems = [p0, p1], [t0, t1], [s0, s1]

    def start(c):
        r0 = row_base + c * _ROWS
        cp = pltpu.async_copy(p_hbm.at[img, pl.ds(r0, _ROWS)], pbufs[c % 2], sems[c % 2])
        ct = pltpu.async_copy(t_hbm.at[img, pl.ds(r0, _ROWS)], tbufs[c % 2], sems[c % 2])
        return cp, ct

    pending = start(0)

    def zbody(r, carry):
        for j in range(_CPAD // _L):
            hist[pl.ds(r * _CPAD + j * _L, _L)] = zeros
        return carry

    lax.fori_loop(0, _K, zbody, 0)

    for c in range(_NCH):
        cp, ct = pending
        cp.wait()
        ct.wait()
        if c + 1 < _NCH:
            pending = start(c + 1)
        pb, tb = pbufs[c % 2], tbufs[c % 2]

        def row_body(r, pb=pb, tb=tb):
            for j in range(_VPR):
                pv = pb[r, pl.ds(j * _L, _L)]
                tv = tb[r, pl.ds(j * _L, _L)]
                plsc.addupdate_scatter(hist, [tv * _CPAD + pv], ones)

        plsc.parallel_loop(0, _ROWS, unroll=2)(row_body)

    def repack(r):
        for j in range(_CPAD // _L):
            hist2d[r, pl.ds(j * _L, _L)] = hist[pl.ds(r * _CPAD + j * _L, _L)]

    plsc.parallel_loop(0, _K)(repack)
    pltpu.sync_copy(hist2d, out_hbm.at[wid])


def _iou_tc(hist_ref, out_ref):
    h = hist_ref[...]
    conf = h.sum(0).astype(jnp.float32)
    rows = lax.broadcasted_iota(jnp.int32, (_K, _CPAD), 0)
    cols = lax.broadcasted_iota(jnp.int32, (_K, _CPAD), 1)
    eye = rows == cols
    diag = jnp.where(eye, conf, 0.0).sum(1, keepdims=True)
    rowsum = conf.sum(1, keepdims=True)
    ct = conf.sum(0, keepdims=True)
    colsum = jnp.where(eye, jnp.broadcast_to(ct, conf.shape), 0.0).sum(
        1, keepdims=True
    )
    iou = diag / (rowsum + colsum - diag)
    out_ref[...] = (iou.sum() / _K).reshape(1, 1)


def kernel(input, target):
    partials = _hist_sc(input, target)
    res = pl.pallas_call(
        _iou_tc,
        out_shape=jax.ShapeDtypeStruct((1, 1), jnp.float32),
    )(partials)
    return res[0, 0]


# speedup vs baseline: 77.8244x; 1.0400x over previous
import functools

import jax
import jax.numpy as jnp
from jax import lax
from jax.experimental import pallas as pl
from jax.experimental.pallas import tpu as pltpu
from jax.experimental.pallas import tpu_sc as plsc

_K = 150
_CPAD = 160
_NC, _NS, _L = 2, 16, 16
_NW = _NC * _NS
_B, _H, _W = 16, 512, 512
_HALF = _H // 2
_ROWS = 32
_NCH = _HALF // _ROWS
_VPR = _W // _L

_mesh = plsc.VectorSubcoreMesh(
    core_axis_name="c", subcore_axis_name="s", num_cores=_NC, num_subcores=_NS
)


@functools.partial(
    pl.kernel,
    out_type=jax.ShapeDtypeStruct((_NW, _K, _CPAD), jnp.int32),
    mesh=_mesh,
    scratch_types=[
        pltpu.VMEM((_K * _CPAD,), jnp.int32),
        pltpu.VMEM((_K, _CPAD), jnp.int32),
        pltpu.VMEM((_ROWS, _W), jnp.int32),
        pltpu.VMEM((_ROWS, _W), jnp.int32),
        pltpu.VMEM((_ROWS, _W), jnp.int32),
        pltpu.VMEM((_ROWS, _W), jnp.int32),
        pltpu.SemaphoreType.DMA,
        pltpu.SemaphoreType.DMA,
    ],
    compiler_params=pltpu.CompilerParams(needs_layout_passes=False),
)
def _hist_sc(p_hbm, t_hbm, out_hbm, hist, hist2d, p0, p1, t0, t1, s0, s1):
    wid = lax.axis_index("s") * _NC + lax.axis_index("c")
    img = wid // 2
    row_base = (wid % 2) * _HALF

    ones = jnp.ones((_L,), jnp.int32)
    zeros = jnp.zeros((_L,), jnp.int32)
    pbufs, tbufs, sems = [p0, p1], [t0, t1], [s0, s1]

    def start(c):
        r0 = row_base + c * _ROWS
        cp = pltpu.async_copy(p_hbm.at[img, pl.ds(r0, _ROWS)], pbufs[c % 2], sems[c % 2])
        ct = pltpu.async_copy(t_hbm.at[img, pl.ds(r0, _ROWS)], tbufs[c % 2], sems[c % 2])
        return cp, ct

    pending = start(0)

    def zbody(r, carry):
        for j in range(_CPAD // _L):
            hist[pl.ds(r * _CPAD + j * _L, _L)] = zeros
        return carry

    lax.fori_loop(0, _K, zbody, 0)

    for c in range(_NCH):
        cp, ct = pending
        cp.wait()
        ct.wait()
        if c + 1 < _NCH:
            pending = start(c + 1)
        pb, tb = pbufs[c % 2], tbufs[c % 2]

        def row_body(r, pb=pb, tb=tb):
            for j in range(_VPR):
                pv = pb[r, pl.ds(j * _L, _L)]
                tv = tb[r, pl.ds(j * _L, _L)]
                plsc.addupdate_scatter(hist, [tv * _CPAD + pv], ones)

        plsc.parallel_loop(0, _ROWS)(row_body)

    def repack(r):
        for j in range(_CPAD // _L):
            hist2d[r, pl.ds(j * _L, _L)] = hist[pl.ds(r * _CPAD + j * _L, _L)]

    plsc.parallel_loop(0, _K)(repack)
    pltpu.sync_copy(hist2d, out_hbm.at[wid])


def _iou_tc(hist_ref, out_ref):
    h = hist_ref[...]
    conf = h.sum(0).astype(jnp.float32)
    rows = lax.broadcasted_iota(jnp.int32, (_K, _CPAD), 0)
    cols = lax.broadcasted_iota(jnp.int32, (_K, _CPAD), 1)
    eye = rows == cols
    diag = jnp.where(eye, conf, 0.0).sum(1, keepdims=True)
    rowsum = conf.sum(1, keepdims=True)
    ct = conf.sum(0, keepdims=True)
    colsum = jnp.where(eye, jnp.broadcast_to(ct, conf.shape), 0.0).sum(
        1, keepdims=True
    )
    iou = diag / (rowsum + colsum - diag)
    out_ref[...] = (iou.sum() / _K).reshape(1, 1)


def kernel(input, target):
    partials = _hist_sc(input, target)
    res = pl.pallas_call(
        _iou_tc,
        out_shape=jax.ShapeDtypeStruct((1, 1), jnp.float32),
    )(partials)
    return res[0, 0]


# speedup vs baseline: 91.0552x; 1.1700x over previous
import functools

import jax
import jax.numpy as jnp
from jax import lax
from jax.experimental import pallas as pl
from jax.experimental.pallas import tpu as pltpu
from jax.experimental.pallas import tpu_sc as plsc

_K = 150
_CPAD = 160
_NC, _NS, _L = 2, 16, 16
_NW = _NC * _NS
_B, _H, _W = 16, 512, 512
_HALF = _H // 2
_ROWS = 32
_NCH = _HALF // _ROWS
_VPR = _W // _L

_mesh = plsc.VectorSubcoreMesh(
    core_axis_name="c", subcore_axis_name="s", num_cores=_NC, num_subcores=_NS
)


@functools.partial(
    pl.kernel,
    out_type=jax.ShapeDtypeStruct((_NW, _K, _CPAD), jnp.int32),
    mesh=_mesh,
    scratch_types=[
        pltpu.VMEM((_K * _CPAD,), jnp.int32),
        pltpu.VMEM((_K, _CPAD), jnp.int32),
        pltpu.VMEM((_ROWS, _W), jnp.int32),
        pltpu.VMEM((_ROWS, _W), jnp.int32),
        pltpu.VMEM((_ROWS, _W), jnp.int32),
        pltpu.VMEM((_ROWS, _W), jnp.int32),
        pltpu.SemaphoreType.DMA,
        pltpu.SemaphoreType.DMA,
    ],
    compiler_params=pltpu.CompilerParams(needs_layout_passes=False),
)
def _hist_sc(p_hbm, t_hbm, out_hbm, hist, hist2d, p0, p1, t0, t1, s0, s1):
    wid = lax.axis_index("s") * _NC + lax.axis_index("c")
    img = wid // 2
    row_base = (wid % 2) * _HALF

    ones = jnp.ones((_L,), jnp.int32)
    zeros = jnp.zeros((_L,), jnp.int32)
    pbufs, tbufs, sems = [p0, p1], [t0, t1], [s0, s1]

    def start(c, buf):
        r0 = row_base + c * _ROWS
        pltpu.async_copy(p_hbm.at[img, pl.ds(r0, _ROWS)], pbufs[buf], sems[buf])
        pltpu.async_copy(t_hbm.at[img, pl.ds(r0, _ROWS)], tbufs[buf], sems[buf])

    def wait_chunk(c, buf):
        r0 = row_base + c * _ROWS
        pltpu.make_async_copy(p_hbm.at[img, pl.ds(r0, _ROWS)], pbufs[buf], sems[buf]).wait()
        pltpu.make_async_copy(t_hbm.at[img, pl.ds(r0, _ROWS)], tbufs[buf], sems[buf]).wait()

    def process(pb, tb):
        def row_body(r):
            for j in range(_VPR):
                pv = pb[r, pl.ds(j * _L, _L)]
                tv = tb[r, pl.ds(j * _L, _L)]
                plsc.addupdate_scatter(hist, [tv * _CPAD + pv], ones)

        plsc.parallel_loop(0, _ROWS)(row_body)

    start(0, 0)

    def zbody(r, carry):
        for j in range(_CPAD // _L):
            hist[pl.ds(r * _CPAD + j * _L, _L)] = zeros
        return carry

    lax.fori_loop(0, _K, zbody, 0)

    def pair_body(c2, carry):
        ca = 2 * c2
        wait_chunk(ca, 0)
        start(ca + 1, 1)
        process(pbufs[0], tbufs[0])
        wait_chunk(ca + 1, 1)

        @pl.when(c2 < _NCH // 2 - 1)
        def _():
            start(ca + 2, 0)

        process(pbufs[1], tbufs[1])
        return carry

    lax.fori_loop(0, _NCH // 2, pair_body, 0)

    def repack(r):
        for j in range(_CPAD // _L):
            hist2d[r, pl.ds(j * _L, _L)] = hist[pl.ds(r * _CPAD + j * _L, _L)]

    plsc.parallel_loop(0, _K)(repack)
    pltpu.sync_copy(hist2d, out_hbm.at[wid])


def _iou_tc(hist_ref, out_ref):
    h = hist_ref[...]
    conf = h.sum(0).astype(jnp.float32)
    rows = lax.broadcasted_iota(jnp.int32, (_K, _CPAD), 0)
    cols = lax.broadcasted_iota(jnp.int32, (_K, _CPAD), 1)
    eye = rows == cols
    diag = jnp.where(eye, conf, 0.0).sum(1, keepdims=True)
    rowsum = conf.sum(1, keepdims=True)
    ct = conf.sum(0, keepdims=True)
    colsum = jnp.where(eye, jnp.broadcast_to(ct, conf.shape), 0.0).sum(
        1, keepdims=True
    )
    iou = diag / (rowsum + colsum - diag)
    out_ref[...] = (iou.sum() / _K).reshape(1, 1)


def kernel(input, target):
    partials = _hist_sc(input, target)
    res = pl.pallas_call(
        _iou_tc,
        out_shape=jax.ShapeDtypeStruct((1, 1), jnp.float32),
    )(partials)
    return res[0, 0]
